# Initial kernel scaffold; baseline (speedup 1.0000x reference)
import jax
import jax.numpy as jnp
from jax.experimental import pallas as pl


def kernel(x, edge_index, batch, W1, b1, W2, b2, Wp, bp):
    raise NotImplementedError("write your pallas kernel here")


# speedup vs baseline: 8.5515x; 8.5515x over previous
import functools

import jax
import jax.numpy as jnp
from jax import lax
from jax.experimental import pallas as pl
from jax.experimental.pallas import tpu as pltpu
from jax.experimental.pallas import tpu_sc as plsc

N = 10000
G = 64
EMB = 256
NP = 10240
DUMP = 10000
P_ITEMS = 172032
ITEM_ROWS = P_ITEMS // 128
BLK_P1 = ITEM_ROWS // 32
BLK_P3 = ITEM_ROWS // 16
ROWS_T = NP // 16
HALF = 128

_mesh = plsc.VectorSubcoreMesh(core_axis_name="c", subcore_axis_name="s")


def _f32(shape):
    return jax.ShapeDtypeStruct(shape, jnp.float32)


def _p1_body(dst_hbm, degp, dstb, ones_t, zbuf, dacc):
    c = lax.axis_index("c")
    s = lax.axis_index("s")
    w = c * 16 + s
    pltpu.sync_copy(dst_hbm.at[w], dstb)
    one = jnp.full((16,), 1.0, jnp.float32)
    zero = jnp.zeros((16,), jnp.float32)
    for j in range(8):
        ones_t[0, pl.ds(j * 16, 16)] = one
    for j in range(40):
        zbuf[pl.ds(j * 16, 16)] = zero
    pltpu.sync_copy(zbuf, dacc.at[pl.ds(s * 640, 640)])
    plsc.subcore_barrier()

    def blk(b, carry):
        pltpu.sync_copy(ones_t.at[0], dacc.at[dstb.at[b, 0]], add=True)
        return carry

    lax.fori_loop(0, BLK_P1, blk, 0)
    plsc.subcore_barrier()
    pltpu.sync_copy(dacc.at[pl.ds(s * 640, 640)], degp.at[c, s, 0])


_p1 = pl.kernel(
    _p1_body,
    out_type=_f32((2, 16, 1, 640)),
    mesh=_mesh,
    scratch_types=[
        pltpu.VMEM((BLK_P1, 1, 128), jnp.int32),
        pltpu.VMEM((1, 128), jnp.float32),
        pltpu.VMEM((640,), jnp.float32),
        pltpu.VMEM_SHARED((NP,), jnp.float32),
    ],
)


def _p2_body(x_ref, w1_ref, degp_ref, hs_ref, dinv_ref):
    dd = degp_ref[0] + degp_ref[1]
    dinvb = lax.rsqrt(jnp.maximum(dd, 1.0))
    dinv_ref[...] = dinvb
    h = jnp.dot(x_ref[...], w1_ref[...], preferred_element_type=jnp.float32)
    hs_ref[...] = h * dinvb


def _p2(xpad, W1, degp3):
    return pl.pallas_call(
        _p2_body,
        grid=(10,),
        in_specs=[
            pl.BlockSpec((1024, EMB), lambda i: (i, 0)),
            pl.BlockSpec((EMB, EMB), lambda i: (0, 0)),
            pl.BlockSpec((2, 1024, 1), lambda i: (0, i, 0)),
        ],
        out_specs=[
            pl.BlockSpec((1024, EMB), lambda i: (i, 0)),
            pl.BlockSpec((1024, 1), lambda i: (i, 0)),
        ],
        out_shape=[_f32((NP, EMB)), _f32((NP, 1))],
    )(xpad, W1, degp3)


def _p3_body(hs2, srcr, dstr, dinv_h, b1_h, batch_h,
             ssum, h1s,
             dinv_t, srcb, dstb, idxb, widx, rows_v, zrow, bkt, bid2,
             b1_t, batch_t, acc, sbkt):
    c = lax.axis_index("c")
    s = lax.axis_index("s")
    base = s * ROWS_T
    i16 = lax.iota(jnp.int32, 16)
    zero = jnp.zeros((16,), jnp.float32)

    pltpu.sync_copy(dinv_h.at[pl.ds(base, ROWS_T)], dinv_t.at[pl.ds(0, ROWS_T)])
    pltpu.sync_copy(b1_h.at[c, 0], b1_t)
    pltpu.sync_copy(batch_h.at[s], batch_t)
    for r in range(16):
        for j in range(8):
            zrow[r, pl.ds(j * 16, 16)] = zero

    def zbkt(r, carry):
        for j in range(8):
            bkt[r, pl.ds(j * 16, 16)] = zero
        return carry

    lax.fori_loop(0, G + 16, zbkt, 0)
    for j in range(5):
        bid2[0, pl.ds(j * 16, 16)] = i16 + j * 16
    for q in range(ROWS_T // 16):
        pltpu.sync_copy(zrow, acc.at[pl.ds(base + q * 16, 16)])

    @pl.when(s == 0)
    def _():
        for q in range(5):
            pltpu.sync_copy(zrow, sbkt.at[pl.ds(q * 16, 16)])

    plsc.subcore_barrier()

    def scatter_pass(gsrc):
        def scat(b, carry):
            for j in range(8):
                sj = srcb[b, 0, pl.ds(j * 16, 16)]
                idxb[pl.ds(j * 16, 16)] = sj * 2 + c
            pltpu.sync_copy(gsrc.at[idxb], rows_v)
            pltpu.sync_copy(rows_v, acc.at[dstb.at[b, 0]], add=True)
            return carry

        for h in range(2):
            pltpu.sync_copy(srcr.at[s, pl.ds(h * 42, 42)], srcb)
            pltpu.sync_copy(dstr.at[s, pl.ds(h * 42, 42)], dstb)
            lax.fori_loop(0, 42, scat, 0)

    scatter_pass(hs2)
    plsc.subcore_barrier()

    b1v = [b1_t[pl.ds(j * 16, 16)] for j in range(8)]

    def epiA_row(r, cc_loc):
        d = dinv_t[pl.ds(cc_loc + r, 16)][0]
        for j in range(8):
            v = rows_v[r, pl.ds(j * 16, 16)]
            v = jnp.maximum(v * d + b1v[j], 0.0) * d
            rows_v[r, pl.ds(j * 16, 16)] = v
        return cc_loc

    for cc in range(5):
        cc_base = base + cc * 128
        pltpu.sync_copy(acc.at[pl.ds(cc_base, 128)], rows_v)
        lax.fori_loop(0, 128, epiA_row, cc * 128)
        for j in range(8):
            widx[0, pl.ds(j * 16, 16)] = (i16 + (cc_base + j * 16)) * 2 + c
        pltpu.sync_copy(rows_v, h1s.at[widx.at[0]])
        for q in range(8):
            pltpu.sync_copy(zrow, acc.at[pl.ds(cc_base + q * 16, 16)])

    plsc.subcore_barrier()

    scatter_pass(h1s)
    plsc.subcore_barrier()

    def epiB_grp(g8, cc):
        r0 = g8 * 16
        gvec = batch_t[cc, 0, pl.ds(r0, 16)]
        dvec = dinv_t[pl.ds(cc * 128 + r0, 16)]
        for k in range(16):
            g = gvec[k]
            d = dvec[k]
            r = r0 + k
            for j in range(8):
                v = rows_v[r, pl.ds(j * 16, 16)] * d
                bkt[g, pl.ds(j * 16, 16)] = bkt[g, pl.ds(j * 16, 16)] + v
        return cc

    def epiB_cc(cc, carry):
        pltpu.sync_copy(acc.at[pl.ds(base + cc * 128, 128)], rows_v)
        lax.fori_loop(0, 8, epiB_grp, cc)
        return carry

    lax.fori_loop(0, 5, epiB_cc, 0)

    pltpu.sync_copy(bkt, sbkt.at[bid2.at[0]], add=True)
    plsc.subcore_barrier()

    @pl.when(s == 0)
    def _():
        pltpu.sync_copy(sbkt, ssum.at[c])


_p3 = pl.kernel(
    _p3_body,
    out_type=(_f32((2, G + 16, HALF)), _f32((NP * 2, HALF))),
    mesh=_mesh,
    scratch_types=[
        pltpu.VMEM((ROWS_T + 16,), jnp.float32),
        pltpu.VMEM((42, 1, 128), jnp.int32),
        pltpu.VMEM((42, 1, 128), jnp.int32),
        pltpu.VMEM((128,), jnp.int32),
        pltpu.VMEM((1, 128), jnp.int32),
        pltpu.VMEM((128, HALF), jnp.float32),
        pltpu.VMEM((16, HALF), jnp.float32),
        pltpu.VMEM((G + 16, HALF), jnp.float32),
        pltpu.VMEM((1, G + 16), jnp.int32),
        pltpu.VMEM((HALF,), jnp.float32),
        pltpu.VMEM((5, 1, 128), jnp.int32),
        pltpu.VMEM_SHARED((NP, HALF), jnp.float32),
        pltpu.VMEM_SHARED((G + 16, HALF), jnp.float32),
    ],
)


def _p4_body(ssum_ref, batch_ref, w2_ref, b2_ref, wp_ref, bp_ref, out_ref):
    batchv = batch_ref[...]
    gids = lax.broadcasted_iota(jnp.int32, (G, NP // 128, 128), 0)
    eq = (batchv[None, :, :] == gids).astype(jnp.float32)
    cnt = jnp.sum(eq, axis=(1, 2))
    S = jnp.concatenate([ssum_ref[0, :G, :], ssum_ref[1, :G, :]], axis=1)
    Sm = S / jnp.maximum(cnt, 1.0)[:, None]
    E2 = jnp.dot(Sm, w2_ref[...], preferred_element_type=jnp.float32) + b2_ref[...]
    out_ref[...] = jnp.dot(E2, wp_ref[...], preferred_element_type=jnp.float32) + bp_ref[...]


def _p4(ssum, batch2d, W2, b2, Wp, bp):
    return pl.pallas_call(
        _p4_body,
        out_shape=_f32((G, 16)),
    )(ssum, batch2d, W2, b2.reshape(1, EMB), Wp, bp.reshape(1, 16))


@jax.jit
def kernel(x, edge_index, batch, W1, b1, W2, b2, Wp, bp):
    src = edge_index[0].astype(jnp.int32)
    dst = edge_index[1].astype(jnp.int32)
    loop = jnp.arange(N, dtype=jnp.int32)
    npad = P_ITEMS - src.shape[0] - N
    fill = jnp.full((npad,), DUMP, jnp.int32)
    src_all = jnp.concatenate([src, loop, fill]).reshape(16, BLK_P3, 1, 128)
    dst_all = jnp.concatenate([dst, loop, fill]).reshape(16, BLK_P3, 1, 128)
    batch_pad = jnp.concatenate(
        [batch.astype(jnp.int32), jnp.full((NP - N,), G, jnp.int32)]
    ).reshape(16, 5, 1, 128)
    xpad = jnp.pad(x, ((0, NP - N), (0, 0)))

    degp = _p1(dst_all.reshape(32, BLK_P1, 1, 128))
    Hs, dinv = _p2(xpad, W1, degp.reshape(2, NP, 1))
    hs2 = Hs.reshape(NP * 2, HALF)
    ssum, _h1s = _p3(hs2, src_all, dst_all, dinv.reshape(NP),
                     b1.reshape(2, 1, HALF), batch_pad)
    return _p4(ssum, batch_pad.reshape(NP // 128, 128), W2, b2, Wp, bp)


# speedup vs baseline: 9.3790x; 1.0968x over previous
import jax
import jax.numpy as jnp
from jax import lax
from jax.experimental import pallas as pl
from jax.experimental.pallas import tpu as pltpu
from jax.experimental.pallas import tpu_sc as plsc

N = 10000
G = 64
EMB = 256
NP = 10240
DUMP = 10000
P_ITEMS = 172032
BLK_P1 = P_ITEMS // 32 // 128
NBLK = P_ITEMS // 16 // 64
ROWS_T = NP // 16
HALF = 128
GB = 80

_mesh = plsc.VectorSubcoreMesh(core_axis_name="c", subcore_axis_name="s")


def _f32(shape):
    return jax.ShapeDtypeStruct(shape, jnp.float32)


def _p1_body(dst_hbm, degp, dstb, ones_t, zbuf, dacc):
    c = lax.axis_index("c")
    s = lax.axis_index("s")
    w = c * 16 + s
    pltpu.sync_copy(dst_hbm.at[w], dstb)
    one = jnp.full((16,), 1.0, jnp.float32)
    zero = jnp.zeros((16,), jnp.float32)
    for j in range(8):
        ones_t[0, pl.ds(j * 16, 16)] = one
    for j in range(40):
        zbuf[pl.ds(j * 16, 16)] = zero
    pltpu.sync_copy(zbuf, dacc.at[pl.ds(s * 640, 640)])
    plsc.subcore_barrier()

    def blk(b, carry):
        pltpu.sync_copy(ones_t.at[0], dacc.at[dstb.at[b, 0]], add=True)
        return carry

    lax.fori_loop(0, BLK_P1, blk, 0)
    plsc.subcore_barrier()
    pltpu.sync_copy(dacc.at[pl.ds(s * 640, 640)], degp.at[c, s, 0])


_p1 = pl.kernel(
    _p1_body,
    out_type=_f32((2, 16, 1, 640)),
    mesh=_mesh,
    scratch_types=[
        pltpu.VMEM((BLK_P1, 1, 128), jnp.int32),
        pltpu.VMEM((1, 128), jnp.float32),
        pltpu.VMEM((640,), jnp.float32),
        pltpu.VMEM_SHARED((NP,), jnp.float32),
    ],
)


def _p2_body(x_ref, w1_ref, degp_ref, hs_ref, dinv_ref):
    dd = degp_ref[0] + degp_ref[1]
    dinvb = lax.rsqrt(jnp.maximum(dd, 1.0))
    dinv_ref[...] = dinvb
    h = jnp.dot(x_ref[...], w1_ref[...], preferred_element_type=jnp.float32)
    hs_ref[...] = h * dinvb


def _p2(xpad, W1, degp3):
    return pl.pallas_call(
        _p2_body,
        grid=(10,),
        in_specs=[
            pl.BlockSpec((1024, EMB), lambda i: (i, 0)),
            pl.BlockSpec((EMB, EMB), lambda i: (0, 0)),
            pl.BlockSpec((2, 1024, 1), lambda i: (0, i, 0)),
        ],
        out_specs=[
            pl.BlockSpec((1024, EMB), lambda i: (i, 0)),
            pl.BlockSpec((1024, 1), lambda i: (i, 0)),
        ],
        out_shape=[_f32((NP, EMB)), _f32((NP, 1))],
    )(xpad, W1, degp3)


def _p3_body(hs2, idxr, dstr, dinv_h, b1_h, batch_h,
             ssum, h1s,
             dinv_t, idxcb, dstb, widx, rowsA, rowsB, zrow,
             b1_t, batch_t, acc, sbkt, gsA, gsB, ssA, ssB):
    c = lax.axis_index("c")
    s = lax.axis_index("s")
    base = s * ROWS_T
    i16 = lax.iota(jnp.int32, 16)
    zero = jnp.zeros((16,), jnp.float32)

    pltpu.sync_copy(dinv_h.at[pl.ds(base, ROWS_T)], dinv_t.at[pl.ds(0, ROWS_T)])
    pltpu.sync_copy(b1_h.at[c, 0], b1_t)
    pltpu.sync_copy(batch_h.at[s], batch_t)
    for r in range(16):
        for j in range(8):
            zrow[r, pl.ds(j * 16, 16)] = zero
    for q in range(ROWS_T // 16):
        pltpu.sync_copy(zrow, acc.at[pl.ds(base + q * 16, 16)])

    @pl.when(s == 0)
    def _():
        for q in range(GB // 16):
            pltpu.sync_copy(zrow, sbkt.at[pl.ds(q * 16, 16)])

    plsc.subcore_barrier()

    HB = NBLK // 2

    def scatter_pass(gsrc):
        for hh in range(2):
            pltpu.sync_copy(idxr.at[c, s, hh], idxcb)
            pltpu.sync_copy(dstr.at[s, hh], dstb)
            pltpu.async_copy(gsrc.at[idxcb.at[0, 0]], rowsA, gsA)

            def pair(p, carry):
                bA = 2 * p
                bB = 2 * p + 1

                @pl.when(p > 0)
                def _():
                    pltpu.make_async_copy(rowsB, acc.at[dstb.at[bA - 1, 0]], ssB).wait()

                pltpu.async_copy(gsrc.at[idxcb.at[bB, 0]], rowsB, gsB)
                pltpu.make_async_copy(gsrc.at[idxcb.at[bA, 0]], rowsA, gsA).wait()
                pltpu.async_copy(rowsA, acc.at[dstb.at[bA, 0]], ssA, add=True)
                pltpu.make_async_copy(gsrc.at[idxcb.at[bB, 0]], rowsB, gsB).wait()
                pltpu.async_copy(rowsB, acc.at[dstb.at[bB, 0]], ssB, add=True)

                @pl.when(p < HB // 2 - 1)
                def _():
                    pltpu.make_async_copy(rowsA, acc.at[dstb.at[bA, 0]], ssA).wait()
                    pltpu.async_copy(gsrc.at[idxcb.at[bA + 2, 0]], rowsA, gsA)

                return carry

            lax.fori_loop(0, HB // 2, pair, 0)
            pltpu.make_async_copy(rowsA, acc.at[dstb.at[HB - 2, 0]], ssA).wait()
            pltpu.make_async_copy(rowsB, acc.at[dstb.at[HB - 1, 0]], ssB).wait()

    scatter_pass(hs2)
    plsc.subcore_barrier()

    b1v = [b1_t[pl.ds(j * 16, 16)] for j in range(8)]

    def epiA_row(r, loc0):
        d = dinv_t[pl.ds(loc0 + r, 16)][0]
        for j in range(8):
            v = rowsA[r, pl.ds(j * 16, 16)]
            v = jnp.maximum(v * d + b1v[j], 0.0) * d
            rowsA[r, pl.ds(j * 16, 16)] = v
        return loc0

    for u in range(10):
        r0 = base + u * 64
        pltpu.sync_copy(acc.at[pl.ds(r0, 64)], rowsA)
        lax.fori_loop(0, 64, epiA_row, u * 64)
        for j in range(4):
            widx[0, pl.ds(j * 16, 16)] = (i16 + (r0 + j * 16)) * 2 + c
        pltpu.sync_copy(rowsA, h1s.at[widx.at[0]])
        for q in range(4):
            pltpu.sync_copy(zrow, acc.at[pl.ds(r0 + q * 16, 16)])

    plsc.subcore_barrier()

    scatter_pass(h1s)
    plsc.subcore_barrier()

    def epiB_grp(g16, loc0):
        rr0 = g16 * 16
        dvec = dinv_t[pl.ds(loc0 + rr0, 16)]
        for k in range(16):
            d = dvec[k]
            r = rr0 + k
            for j in range(8):
                rowsA[r, pl.ds(j * 16, 16)] = rowsA[r, pl.ds(j * 16, 16)] * d
        return loc0

    for u in range(10):
        r0 = base + u * 64
        pltpu.sync_copy(acc.at[pl.ds(r0, 64)], rowsA)
        lax.fori_loop(0, 4, epiB_grp, u * 64)
        pltpu.sync_copy(rowsA, sbkt.at[batch_t.at[u, 0]], add=True)

    plsc.subcore_barrier()

    @pl.when(s == 0)
    def _():
        pltpu.sync_copy(sbkt, ssum.at[c])


_p3 = pl.kernel(
    _p3_body,
    out_type=(_f32((2, GB, HALF)), _f32((NP * 2, HALF))),
    mesh=_mesh,
    scratch_types=[
        pltpu.VMEM((ROWS_T + 16,), jnp.float32),
        pltpu.VMEM((NBLK // 2, 1, 64), jnp.int32),
        pltpu.VMEM((NBLK // 2, 1, 64), jnp.int32),
        pltpu.VMEM((1, 64), jnp.int32),
        pltpu.VMEM((64, HALF), jnp.float32),
        pltpu.VMEM((64, HALF), jnp.float32),
        pltpu.VMEM((16, HALF), jnp.float32),
        pltpu.VMEM((HALF,), jnp.float32),
        pltpu.VMEM((10, 1, 64), jnp.int32),
        pltpu.VMEM_SHARED((NP, HALF), jnp.float32),
        pltpu.VMEM_SHARED((GB, HALF), jnp.float32),
        pltpu.SemaphoreType.DMA,
        pltpu.SemaphoreType.DMA,
        pltpu.SemaphoreType.DMA,
        pltpu.SemaphoreType.DMA,
    ],
)


def _p4_body(ssum_ref, batch_ref, w2_ref, b2_ref, wp_ref, bp_ref, out_ref):
    batchv = batch_ref[...]
    gids = lax.broadcasted_iota(jnp.int32, (G, NP // 128, 128), 0)
    eq = (batchv[None, :, :] == gids).astype(jnp.float32)
    cnt = jnp.sum(eq, axis=(1, 2))
    S = jnp.concatenate([ssum_ref[0, :G, :], ssum_ref[1, :G, :]], axis=1)
    Sm = S / jnp.maximum(cnt, 1.0)[:, None]
    E2 = jnp.dot(Sm, w2_ref[...], preferred_element_type=jnp.float32) + b2_ref[...]
    out_ref[...] = jnp.dot(E2, wp_ref[...], preferred_element_type=jnp.float32) + bp_ref[...]


def _p4(ssum, batch2d, W2, b2, Wp, bp):
    return pl.pallas_call(
        _p4_body,
        out_shape=_f32((G, 16)),
    )(ssum, batch2d, W2, b2.reshape(1, EMB), Wp, bp.reshape(1, 16))


@jax.jit
def kernel(x, edge_index, batch, W1, b1, W2, b2, Wp, bp):
    src = edge_index[0].astype(jnp.int32)
    dst = edge_index[1].astype(jnp.int32)
    loop = jnp.arange(N, dtype=jnp.int32)
    npad = P_ITEMS - src.shape[0] - N
    fill = jnp.full((npad,), DUMP, jnp.int32)
    src_all = jnp.concatenate([src, loop, fill])
    dst_all = jnp.concatenate([dst, loop, fill])
    src2 = src_all * 2
    idxr = jnp.stack([src2, src2 + 1]).reshape(2, 16, 2, NBLK // 2, 1, 64)
    dstr = dst_all.reshape(16, 2, NBLK // 2, 1, 64)
    batch_pad = jnp.concatenate(
        [batch.astype(jnp.int32), jnp.full((NP - N,), G, jnp.int32)]
    )
    xpad = jnp.pad(x, ((0, NP - N), (0, 0)))

    degp = _p1(dst_all.reshape(32, BLK_P1, 1, 128))
    Hs, dinv = _p2(xpad, W1, degp.reshape(2, NP, 1))
    hs2 = Hs.reshape(NP * 2, HALF)
    ssum, _h1s = _p3(hs2, idxr, dstr, dinv.reshape(NP),
                     b1.reshape(2, 1, HALF), batch_pad.reshape(16, 10, 1, 64))
    return _p4(ssum, batch_pad.reshape(NP // 128, 128), W2, b2, Wp, bp)


# speedup vs baseline: 9.6452x; 1.0284x over previous
import jax
import jax.numpy as jnp
from jax import lax
from jax.experimental import pallas as pl
from jax.experimental.pallas import tpu as pltpu
from jax.experimental.pallas import tpu_sc as plsc

N = 10000
G = 64
EMB = 256
NP = 10240
DUMP = 10000
P_ITEMS = 172032
BLK_P1 = P_ITEMS // 32 // 128
BLK = 96
NBLK = P_ITEMS // 16 // BLK
NST = 8
SB = NBLK // NST
ROWS_T = NP // 16
HALF = 128
GB = 80

_mesh = plsc.VectorSubcoreMesh(core_axis_name="c", subcore_axis_name="s")


def _f32(shape):
    return jax.ShapeDtypeStruct(shape, jnp.float32)


def _p1_body(dst_hbm, degp, dstb, ones_t, zbuf, dacc):
    c = lax.axis_index("c")
    s = lax.axis_index("s")
    w = c * 16 + s
    pltpu.sync_copy(dst_hbm.at[w], dstb)
    one = jnp.full((16,), 1.0, jnp.float32)
    zero = jnp.zeros((16,), jnp.float32)
    for j in range(8):
        ones_t[0, pl.ds(j * 16, 16)] = one
    for j in range(40):
        zbuf[pl.ds(j * 16, 16)] = zero
    pltpu.sync_copy(zbuf, dacc.at[pl.ds(s * 640, 640)])
    plsc.subcore_barrier()

    def blk(b, carry):
        pltpu.sync_copy(ones_t.at[0], dacc.at[dstb.at[b, 0]], add=True)
        return carry

    lax.fori_loop(0, BLK_P1, blk, 0)
    plsc.subcore_barrier()
    pltpu.sync_copy(dacc.at[pl.ds(s * 640, 640)], degp.at[c, s, 0])


_p1 = pl.kernel(
    _p1_body,
    out_type=_f32((2, 16, 1, 640)),
    mesh=_mesh,
    scratch_types=[
        pltpu.VMEM((BLK_P1, 1, 128), jnp.int32),
        pltpu.VMEM((1, 128), jnp.float32),
        pltpu.VMEM((640,), jnp.float32),
        pltpu.VMEM_SHARED((NP,), jnp.float32),
    ],
)


def _p2_body(x_ref, w1_ref, degp_ref, hs_ref, dinv_ref):
    dd = degp_ref[0] + degp_ref[1]
    dinvb = lax.rsqrt(jnp.maximum(dd, 1.0))
    dinv_ref[...] = dinvb
    h = jnp.dot(x_ref[...], w1_ref[...], preferred_element_type=jnp.float32)
    hs_ref[...] = h * dinvb


def _p2(xpad, W1, degp3):
    return pl.pallas_call(
        _p2_body,
        grid=(10,),
        in_specs=[
            pl.BlockSpec((1024, EMB), lambda i: (i, 0)),
            pl.BlockSpec((EMB, EMB), lambda i: (0, 0)),
            pl.BlockSpec((2, 1024, 1), lambda i: (0, i, 0)),
        ],
        out_specs=[
            pl.BlockSpec((1024, EMB), lambda i: (i, 0)),
            pl.BlockSpec((1024, 1), lambda i: (i, 0)),
        ],
        out_shape=[_f32((NP, EMB)), _f32((NP, 1))],
    )(xpad, W1, degp3)


def _p3_body(hs2, idxr, dstr, dinv_h, b1_h, batch_h,
             ssum, h1s,
             dinv_t, idxcb, dstb, widx, rowsA, rowsB, zrow,
             b1_t, batch_t, acc, sbkt, gsA, gsB, ssA, ssB):
    c = lax.axis_index("c")
    s = lax.axis_index("s")
    base = s * ROWS_T
    i16 = lax.iota(jnp.int32, 16)
    zero = jnp.zeros((16,), jnp.float32)

    pltpu.sync_copy(dinv_h.at[pl.ds(base, ROWS_T)], dinv_t.at[pl.ds(0, ROWS_T)])
    pltpu.sync_copy(b1_h.at[c, 0], b1_t)
    pltpu.sync_copy(batch_h.at[s], batch_t)
    for r in range(16):
        for j in range(8):
            zrow[r, pl.ds(j * 16, 16)] = zero
    for q in range(ROWS_T // 16):
        pltpu.sync_copy(zrow, acc.at[pl.ds(base + q * 16, 16)])

    @pl.when(s == 0)
    def _():
        for q in range(GB // 16):
            pltpu.sync_copy(zrow, sbkt.at[pl.ds(q * 16, 16)])

    plsc.subcore_barrier()

    def scatter_pass(gsrc):
        for hh in range(NST):
            pltpu.sync_copy(idxr.at[c, s, hh], idxcb)
            pltpu.sync_copy(dstr.at[s, hh], dstb)
            pltpu.async_copy(gsrc.at[idxcb.at[0, 0]], rowsA, gsA)

            def pair(p, carry):
                bA = 2 * p
                bB = 2 * p + 1

                @pl.when(p > 0)
                def _():
                    pltpu.make_async_copy(rowsB, acc.at[dstb.at[bA - 1, 0]], ssB).wait()

                pltpu.async_copy(gsrc.at[idxcb.at[bB, 0]], rowsB, gsB)
                pltpu.make_async_copy(gsrc.at[idxcb.at[bA, 0]], rowsA, gsA).wait()
                pltpu.async_copy(rowsA, acc.at[dstb.at[bA, 0]], ssA, add=True)
                pltpu.make_async_copy(gsrc.at[idxcb.at[bB, 0]], rowsB, gsB).wait()
                pltpu.async_copy(rowsB, acc.at[dstb.at[bB, 0]], ssB, add=True)

                @pl.when(p < SB // 2 - 1)
                def _():
                    pltpu.make_async_copy(rowsA, acc.at[dstb.at[bA, 0]], ssA).wait()
                    pltpu.async_copy(gsrc.at[idxcb.at[bA + 2, 0]], rowsA, gsA)

                return carry

            lax.fori_loop(0, SB // 2, pair, 0)
            pltpu.make_async_copy(rowsA, acc.at[dstb.at[SB - 2, 0]], ssA).wait()
            pltpu.make_async_copy(rowsB, acc.at[dstb.at[SB - 1, 0]], ssB).wait()

    scatter_pass(hs2)
    plsc.subcore_barrier()

    b1v = [b1_t[pl.ds(j * 16, 16)] for j in range(8)]

    def epiA_row(r, loc0):
        d = dinv_t[pl.ds(loc0 + r, 16)][0]
        for j in range(8):
            v = rowsA[r, pl.ds(j * 16, 16)]
            v = jnp.maximum(v * d + b1v[j], 0.0) * d
            rowsA[r, pl.ds(j * 16, 16)] = v
        return loc0

    for u in range(10):
        r0 = base + u * 64
        pltpu.sync_copy(acc.at[pl.ds(r0, 64)], rowsA.at[pl.ds(0, 64)])
        lax.fori_loop(0, 64, epiA_row, u * 64)
        for j in range(4):
            widx[0, pl.ds(j * 16, 16)] = (i16 + (r0 + j * 16)) * 2 + c
        pltpu.sync_copy(rowsA.at[pl.ds(0, 64)], h1s.at[widx.at[0]])
        for q in range(4):
            pltpu.sync_copy(zrow, acc.at[pl.ds(r0 + q * 16, 16)])

    plsc.subcore_barrier()

    scatter_pass(h1s)
    plsc.subcore_barrier()

    def epiB_grp(g16, loc0):
        rr0 = g16 * 16
        dvec = dinv_t[pl.ds(loc0 + rr0, 16)]
        for k in range(16):
            d = dvec[k]
            r = rr0 + k
            for j in range(8):
                rowsA[r, pl.ds(j * 16, 16)] = rowsA[r, pl.ds(j * 16, 16)] * d
        return loc0

    for u in range(10):
        r0 = base + u * 64
        pltpu.sync_copy(acc.at[pl.ds(r0, 64)], rowsA.at[pl.ds(0, 64)])
        lax.fori_loop(0, 4, epiB_grp, u * 64)
        pltpu.sync_copy(rowsA.at[pl.ds(0, 64)], sbkt.at[batch_t.at[u, 0]], add=True)

    plsc.subcore_barrier()

    @pl.when(s == 0)
    def _():
        pltpu.sync_copy(sbkt, ssum.at[c])


_p3 = pl.kernel(
    _p3_body,
    out_type=(_f32((2, GB, HALF)), _f32((NP * 2, HALF))),
    mesh=_mesh,
    scratch_types=[
        pltpu.VMEM((ROWS_T + 16,), jnp.float32),
        pltpu.VMEM((SB, 1, BLK), jnp.int32),
        pltpu.VMEM((SB, 1, BLK), jnp.int32),
        pltpu.VMEM((1, 64), jnp.int32),
        pltpu.VMEM((BLK, HALF), jnp.float32),
        pltpu.VMEM((BLK, HALF), jnp.float32),
        pltpu.VMEM((16, HALF), jnp.float32),
        pltpu.VMEM((HALF,), jnp.float32),
        pltpu.VMEM((10, 1, 64), jnp.int32),
        pltpu.VMEM_SHARED((NP, HALF), jnp.float32),
        pltpu.VMEM_SHARED((GB, HALF), jnp.float32),
        pltpu.SemaphoreType.DMA,
        pltpu.SemaphoreType.DMA,
        pltpu.SemaphoreType.DMA,
        pltpu.SemaphoreType.DMA,
    ],
)


def _p4_body(ssum_ref, batch_ref, w2_ref, b2_ref, wp_ref, bp_ref, out_ref):
    batchv = batch_ref[...]
    gids = lax.broadcasted_iota(jnp.int32, (G, NP // 128, 128), 0)
    eq = (batchv[None, :, :] == gids).astype(jnp.float32)
    cnt = jnp.sum(eq, axis=(1, 2))
    S = jnp.concatenate([ssum_ref[0, :G, :], ssum_ref[1, :G, :]], axis=1)
    Sm = S / jnp.maximum(cnt, 1.0)[:, None]
    E2 = jnp.dot(Sm, w2_ref[...], preferred_element_type=jnp.float32) + b2_ref[...]
    out_ref[...] = jnp.dot(E2, wp_ref[...], preferred_element_type=jnp.float32) + bp_ref[...]


def _p4(ssum, batch2d, W2, b2, Wp, bp):
    return pl.pallas_call(
        _p4_body,
        out_shape=_f32((G, 16)),
    )(ssum, batch2d, W2, b2.reshape(1, EMB), Wp, bp.reshape(1, 16))


@jax.jit
def kernel(x, edge_index, batch, W1, b1, W2, b2, Wp, bp):
    src = edge_index[0].astype(jnp.int32)
    dst = edge_index[1].astype(jnp.int32)
    loop = jnp.arange(N, dtype=jnp.int32)
    npad = P_ITEMS - src.shape[0] - N
    fill = jnp.full((npad,), DUMP, jnp.int32)
    src_all = jnp.concatenate([src, loop, fill])
    dst_all = jnp.concatenate([dst, loop, fill])
    src2 = src_all * 2
    idxr = jnp.stack([src2, src2 + 1]).reshape(2, 16, NST, SB, 1, BLK)
    dstr = dst_all.reshape(16, NST, SB, 1, BLK)
    batch_pad = jnp.concatenate(
        [batch.astype(jnp.int32), jnp.full((NP - N,), G, jnp.int32)]
    )
    xpad = jnp.pad(x, ((0, NP - N), (0, 0)))

    degp = _p1(dst_all.reshape(32, BLK_P1, 1, 128))
    Hs, dinv = _p2(xpad, W1, degp.reshape(2, NP, 1))
    hs2 = Hs.reshape(NP * 2, HALF)
    ssum, _h1s = _p3(hs2, idxr, dstr, dinv.reshape(NP),
                     b1.reshape(2, 1, HALF), batch_pad.reshape(16, 10, 1, 64))
    return _p4(ssum, batch_pad.reshape(NP // 128, 128), W2, b2, Wp, bp)


# speedup vs baseline: 10.0544x; 1.0424x over previous
import jax
import jax.numpy as jnp
from jax import lax
from jax.experimental import pallas as pl
from jax.experimental.pallas import tpu as pltpu
from jax.experimental.pallas import tpu_sc as plsc

N = 10000
G = 64
EMB = 256
NP = 10240
DUMP = 10000
P_ITEMS = 172032
BLK_P1 = P_ITEMS // 32 // 128
BLK = 64
NBLK = P_ITEMS // 16 // BLK
NST = 8
SB = NBLK // NST
ROWS_T = NP // 16
HALF = 128
GB = 80

_mesh = plsc.VectorSubcoreMesh(core_axis_name="c", subcore_axis_name="s")


def _f32(shape):
    return jax.ShapeDtypeStruct(shape, jnp.float32)


def _p1_body(dst_hbm, degp, dstb, ones_t, zbuf, dacc):
    c = lax.axis_index("c")
    s = lax.axis_index("s")
    w = c * 16 + s
    pltpu.sync_copy(dst_hbm.at[w], dstb)
    one = jnp.full((16,), 1.0, jnp.float32)
    zero = jnp.zeros((16,), jnp.float32)
    for j in range(8):
        ones_t[0, pl.ds(j * 16, 16)] = one
    for j in range(40):
        zbuf[pl.ds(j * 16, 16)] = zero
    pltpu.sync_copy(zbuf, dacc.at[pl.ds(s * 640, 640)])
    plsc.subcore_barrier()

    def blk(b, carry):
        pltpu.sync_copy(ones_t.at[0], dacc.at[dstb.at[b, 0]], add=True)
        return carry

    lax.fori_loop(0, BLK_P1, blk, 0)
    plsc.subcore_barrier()
    pltpu.sync_copy(dacc.at[pl.ds(s * 640, 640)], degp.at[c, s, 0])


_p1 = pl.kernel(
    _p1_body,
    out_type=_f32((2, 16, 1, 640)),
    mesh=_mesh,
    scratch_types=[
        pltpu.VMEM((BLK_P1, 1, 128), jnp.int32),
        pltpu.VMEM((1, 128), jnp.float32),
        pltpu.VMEM((640,), jnp.float32),
        pltpu.VMEM_SHARED((NP,), jnp.float32),
    ],
)


def _p2_body(x_ref, w1_ref, degp_ref, hs_ref, dinv_ref):
    dd = degp_ref[0] + degp_ref[1]
    dinvb = lax.rsqrt(jnp.maximum(dd, 1.0))
    dinv_ref[...] = dinvb
    h = jnp.dot(x_ref[...], w1_ref[...], preferred_element_type=jnp.float32)
    hs_ref[...] = h * dinvb


def _p2(xpad, W1, degp3):
    return pl.pallas_call(
        _p2_body,
        grid=(10,),
        in_specs=[
            pl.BlockSpec((1024, EMB), lambda i: (i, 0)),
            pl.BlockSpec((EMB, EMB), lambda i: (0, 0)),
            pl.BlockSpec((2, 1024, 1), lambda i: (0, i, 0)),
        ],
        out_specs=[
            pl.BlockSpec((1024, EMB), lambda i: (i, 0)),
            pl.BlockSpec((1024, 1), lambda i: (i, 0)),
        ],
        out_shape=[_f32((NP, EMB)), _f32((NP, 1))],
    )(xpad, W1, degp3)


def _p3_body(hs2, idxr, dstr, dinv_h, b1_h, batch_h,
             ssum, h1s,
             dinv_t, idxcb, dstb, widx, rowsA, rowsB, rowsC, zrow,
             b1_t, batch_t, acc, sbkt, gsA, gsB, gsC, ssA, ssB, ssC):
    c = lax.axis_index("c")
    s = lax.axis_index("s")
    base = s * ROWS_T
    i16 = lax.iota(jnp.int32, 16)
    zero = jnp.zeros((16,), jnp.float32)

    pltpu.sync_copy(dinv_h.at[pl.ds(base, ROWS_T)], dinv_t.at[pl.ds(0, ROWS_T)])
    pltpu.sync_copy(b1_h.at[c, 0], b1_t)
    pltpu.sync_copy(batch_h.at[s], batch_t)
    for r in range(16):
        for j in range(8):
            zrow[r, pl.ds(j * 16, 16)] = zero
    for q in range(ROWS_T // 16):
        pltpu.sync_copy(zrow, acc.at[pl.ds(base + q * 16, 16)])

    @pl.when(s == 0)
    def _():
        for q in range(GB // 16):
            pltpu.sync_copy(zrow, sbkt.at[pl.ds(q * 16, 16)])

    plsc.subcore_barrier()

    def scatter_pass(gsrc):
        NT = SB // 3

        def g_start(b, buf, sem):
            pltpu.async_copy(gsrc.at[idxcb.at[b, 0]], buf, sem)

        def g_wait(b, buf, sem):
            pltpu.make_async_copy(gsrc.at[idxcb.at[b, 0]], buf, sem).wait()

        def s_start(b, buf, sem):
            pltpu.async_copy(buf, acc.at[dstb.at[b, 0]], sem, add=True)

        def s_wait(b, buf, sem):
            pltpu.make_async_copy(buf, acc.at[dstb.at[b, 0]], sem).wait()

        for hh in range(NST):
            pltpu.sync_copy(idxr.at[c, s, hh], idxcb)
            pltpu.sync_copy(dstr.at[s, hh], dstb)
            g_start(0, rowsA, gsA)
            g_start(1, rowsB, gsB)

            def triple(p, carry):
                bA = 3 * p
                bB = 3 * p + 1
                bC = 3 * p + 2

                @pl.when(p > 0)
                def _():
                    s_wait(bC - 3, rowsC, ssC)

                g_start(bC, rowsC, gsC)
                g_wait(bA, rowsA, gsA)
                s_start(bA, rowsA, ssA)
                g_wait(bB, rowsB, gsB)
                s_start(bB, rowsB, ssB)

                @pl.when(p < NT - 1)
                def _():
                    s_wait(bA, rowsA, ssA)
                    g_start(bA + 3, rowsA, gsA)

                g_wait(bC, rowsC, gsC)
                s_start(bC, rowsC, ssC)

                @pl.when(p < NT - 1)
                def _():
                    s_wait(bB, rowsB, ssB)
                    g_start(bB + 3, rowsB, gsB)

                return carry

            lax.fori_loop(0, NT, triple, 0)
            s_wait(SB - 3, rowsA, ssA)
            s_wait(SB - 2, rowsB, ssB)
            s_wait(SB - 1, rowsC, ssC)

    scatter_pass(hs2)
    plsc.subcore_barrier()

    b1v = [b1_t[pl.ds(j * 16, 16)] for j in range(8)]

    def epiA_row(r, loc0):
        d = dinv_t[pl.ds(loc0 + r, 16)][0]
        for j in range(8):
            v = rowsA[r, pl.ds(j * 16, 16)]
            v = jnp.maximum(v * d + b1v[j], 0.0) * d
            rowsA[r, pl.ds(j * 16, 16)] = v
        return loc0

    for u in range(10):
        r0 = base + u * 64
        pltpu.sync_copy(acc.at[pl.ds(r0, 64)], rowsA)
        lax.fori_loop(0, 64, epiA_row, u * 64)
        for j in range(4):
            widx[0, pl.ds(j * 16, 16)] = (i16 + (r0 + j * 16)) * 2 + c
        pltpu.sync_copy(rowsA, h1s.at[widx.at[0]])
        for q in range(4):
            pltpu.sync_copy(zrow, acc.at[pl.ds(r0 + q * 16, 16)])

    plsc.subcore_barrier()

    scatter_pass(h1s)
    plsc.subcore_barrier()

    def epiB_grp(g16, loc0):
        rr0 = g16 * 16
        dvec = dinv_t[pl.ds(loc0 + rr0, 16)]
        for k in range(16):
            d = dvec[k]
            r = rr0 + k
            for j in range(8):
                rowsA[r, pl.ds(j * 16, 16)] = rowsA[r, pl.ds(j * 16, 16)] * d
        return loc0

    for u in range(10):
        r0 = base + u * 64
        pltpu.sync_copy(acc.at[pl.ds(r0, 64)], rowsA)
        lax.fori_loop(0, 4, epiB_grp, u * 64)
        pltpu.sync_copy(rowsA, sbkt.at[batch_t.at[u, 0]], add=True)

    plsc.subcore_barrier()

    @pl.when(s == 0)
    def _():
        pltpu.sync_copy(sbkt, ssum.at[c])


_p3 = pl.kernel(
    _p3_body,
    out_type=(_f32((2, GB, HALF)), _f32((NP * 2, HALF))),
    mesh=_mesh,
    scratch_types=[
        pltpu.VMEM((ROWS_T + 16,), jnp.float32),
        pltpu.VMEM((SB, 1, BLK), jnp.int32),
        pltpu.VMEM((SB, 1, BLK), jnp.int32),
        pltpu.VMEM((1, 64), jnp.int32),
        pltpu.VMEM((BLK, HALF), jnp.float32),
        pltpu.VMEM((BLK, HALF), jnp.float32),
        pltpu.VMEM((BLK, HALF), jnp.float32),
        pltpu.VMEM((16, HALF), jnp.float32),
        pltpu.VMEM((HALF,), jnp.float32),
        pltpu.VMEM((10, 1, 64), jnp.int32),
        pltpu.VMEM_SHARED((NP, HALF), jnp.float32),
        pltpu.VMEM_SHARED((GB, HALF), jnp.float32),
        pltpu.SemaphoreType.DMA,
        pltpu.SemaphoreType.DMA,
        pltpu.SemaphoreType.DMA,
        pltpu.SemaphoreType.DMA,
        pltpu.SemaphoreType.DMA,
        pltpu.SemaphoreType.DMA,
    ],
)


def _p4_body(ssum_ref, batch_ref, w2_ref, b2_ref, wp_ref, bp_ref, out_ref):
    batchv = batch_ref[...]
    gids = lax.broadcasted_iota(jnp.int32, (G, NP // 128, 128), 0)
    eq = (batchv[None, :, :] == gids).astype(jnp.float32)
    cnt = jnp.sum(eq, axis=(1, 2))
    S = jnp.concatenate([ssum_ref[0, :G, :], ssum_ref[1, :G, :]], axis=1)
    Sm = S / jnp.maximum(cnt, 1.0)[:, None]
    E2 = jnp.dot(Sm, w2_ref[...], preferred_element_type=jnp.float32) + b2_ref[...]
    out_ref[...] = jnp.dot(E2, wp_ref[...], preferred_element_type=jnp.float32) + bp_ref[...]


def _p4(ssum, batch2d, W2, b2, Wp, bp):
    return pl.pallas_call(
        _p4_body,
        out_shape=_f32((G, 16)),
    )(ssum, batch2d, W2, b2.reshape(1, EMB), Wp, bp.reshape(1, 16))


@jax.jit
def kernel(x, edge_index, batch, W1, b1, W2, b2, Wp, bp):
    src = edge_index[0].astype(jnp.int32)
    dst = edge_index[1].astype(jnp.int32)
    loop = jnp.arange(N, dtype=jnp.int32)
    npad = P_ITEMS - src.shape[0] - N
    fill = jnp.full((npad,), DUMP, jnp.int32)
    src_all = jnp.concatenate([src, loop, fill])
    dst_all = jnp.concatenate([dst, loop, fill])
    src2 = src_all * 2
    idxr = jnp.stack([src2, src2 + 1]).reshape(2, 16, NST, SB, 1, BLK)
    dstr = dst_all.reshape(16, NST, SB, 1, BLK)
    batch_pad = jnp.concatenate(
        [batch.astype(jnp.int32), jnp.full((NP - N,), G, jnp.int32)]
    )
    xpad = jnp.pad(x, ((0, NP - N), (0, 0)))

    degp = _p1(dst_all.reshape(32, BLK_P1, 1, 128))
    Hs, dinv = _p2(xpad, W1, degp.reshape(2, NP, 1))
    hs2 = Hs.reshape(NP * 2, HALF)
    ssum, _h1s = _p3(hs2, idxr, dstr, dinv.reshape(NP),
                     b1.reshape(2, 1, HALF), batch_pad.reshape(16, 10, 1, 64))
    return _p4(ssum, batch_pad.reshape(NP // 128, 128), W2, b2, Wp, bp)


# speedup vs baseline: 10.3473x; 1.0291x over previous
import jax
import jax.numpy as jnp
from jax import lax
from jax.experimental import pallas as pl
from jax.experimental.pallas import tpu as pltpu
from jax.experimental.pallas import tpu_sc as plsc

N = 10000
G = 64
EMB = 256
NP = 10240
DUMP = 10000
P_ITEMS = 172032
BLK_P1 = P_ITEMS // 32 // 128
BLK = 64
NBLK = P_ITEMS // 16 // BLK
NST = 8
SB = NBLK // NST
ROWS_T = NP // 16
HALF = 128
GB = 80

_mesh = plsc.VectorSubcoreMesh(core_axis_name="c", subcore_axis_name="s")


def _f32(shape):
    return jax.ShapeDtypeStruct(shape, jnp.float32)


def _p1_body(dst_hbm, degp, dstb, ones_t, zbuf, dacc):
    c = lax.axis_index("c")
    s = lax.axis_index("s")
    w = c * 16 + s
    pltpu.sync_copy(dst_hbm.at[w], dstb)
    one = jnp.full((16,), 1.0, jnp.float32)
    zero = jnp.zeros((16,), jnp.float32)
    for j in range(8):
        ones_t[0, pl.ds(j * 16, 16)] = one
    for j in range(40):
        zbuf[pl.ds(j * 16, 16)] = zero
    pltpu.sync_copy(zbuf, dacc.at[pl.ds(s * 640, 640)])
    plsc.subcore_barrier()

    def blk(b, carry):
        pltpu.sync_copy(ones_t.at[0], dacc.at[dstb.at[b, 0]], add=True)
        return carry

    lax.fori_loop(0, BLK_P1, blk, 0)
    plsc.subcore_barrier()
    pltpu.sync_copy(dacc.at[pl.ds(s * 640, 640)], degp.at[c, s, 0])


_p1 = pl.kernel(
    _p1_body,
    out_type=_f32((2, 16, 1, 640)),
    mesh=_mesh,
    scratch_types=[
        pltpu.VMEM((BLK_P1, 1, 128), jnp.int32),
        pltpu.VMEM((1, 128), jnp.float32),
        pltpu.VMEM((640,), jnp.float32),
        pltpu.VMEM_SHARED((NP,), jnp.float32),
    ],
)


def _p2_body(x_ref, w1_ref, degp_ref, hs_ref, dinv_ref):
    dd = degp_ref[0] + degp_ref[1]
    dinvb = lax.rsqrt(jnp.maximum(dd, 1.0))
    dinv_ref[...] = dinvb
    h = jnp.dot(x_ref[...], w1_ref[...], preferred_element_type=jnp.float32)
    hs_ref[...] = h * dinvb


def _p2(xpad, W1, degp3):
    return pl.pallas_call(
        _p2_body,
        grid=(10,),
        in_specs=[
            pl.BlockSpec((1024, EMB), lambda i: (i, 0)),
            pl.BlockSpec((EMB, EMB), lambda i: (0, 0)),
            pl.BlockSpec((2, 1024, 1), lambda i: (0, i, 0)),
        ],
        out_specs=[
            pl.BlockSpec((1024, EMB), lambda i: (i, 0)),
            pl.BlockSpec((1024, 1), lambda i: (i, 0)),
        ],
        out_shape=[_f32((NP, EMB)), _f32((NP, 1))],
    )(xpad, W1, degp3)


def _p3_body(hs2, idxr, dstr, dinv_h, b1_h, batch_h,
             ssum, h1s,
             dinv_t, idxcb, dstb, idxcb2, dstb2, widx, rowsA, rowsB, rowsC, zrow,
             b1_t, batch_t, acc, sbkt, gsA, gsB, gsC, ssA, ssB, ssC, sgI, sgD):
    c = lax.axis_index("c")
    s = lax.axis_index("s")
    base = s * ROWS_T
    i16 = lax.iota(jnp.int32, 16)
    zero = jnp.zeros((16,), jnp.float32)

    pltpu.sync_copy(dinv_h.at[pl.ds(base, ROWS_T)], dinv_t.at[pl.ds(0, ROWS_T)])
    pltpu.sync_copy(b1_h.at[c, 0], b1_t)
    pltpu.sync_copy(batch_h.at[s], batch_t)
    for r in range(16):
        for j in range(8):
            zrow[r, pl.ds(j * 16, 16)] = zero
    for q in range(ROWS_T // 16):
        pltpu.sync_copy(zrow, acc.at[pl.ds(base + q * 16, 16)])

    @pl.when(s == 0)
    def _():
        for q in range(GB // 16):
            pltpu.sync_copy(zrow, sbkt.at[pl.ds(q * 16, 16)])

    plsc.subcore_barrier()

    def scatter_pass(gsrc):
        NT = SB // 3

        def g_start2(icb, b, buf, sem):
            pltpu.async_copy(gsrc.at[icb.at[b, 0]], buf, sem)

        def g_wait2(icb, b, buf, sem):
            pltpu.make_async_copy(gsrc.at[icb.at[b, 0]], buf, sem).wait()

        def s_start2(dcb, b, buf, sem):
            pltpu.async_copy(buf, acc.at[dcb.at[b, 0]], sem, add=True)

        def s_wait2(dcb, b, buf, sem):
            pltpu.make_async_copy(buf, acc.at[dcb.at[b, 0]], sem).wait()

        def run_chunk(icb, dcb):
            g_start2(icb, 0, rowsA, gsA)
            g_start2(icb, 1, rowsB, gsB)

            def triple(p, carry):
                bA = 3 * p
                bB = 3 * p + 1
                bC = 3 * p + 2

                @pl.when(p > 0)
                def _():
                    s_wait2(dcb, bC - 3, rowsC, ssC)

                g_start2(icb, bC, rowsC, gsC)
                g_wait2(icb, bA, rowsA, gsA)
                s_start2(dcb, bA, rowsA, ssA)
                g_wait2(icb, bB, rowsB, gsB)
                s_start2(dcb, bB, rowsB, ssB)

                @pl.when(p < NT - 1)
                def _():
                    s_wait2(dcb, bA, rowsA, ssA)
                    g_start2(icb, bA + 3, rowsA, gsA)

                g_wait2(icb, bC, rowsC, gsC)
                s_start2(dcb, bC, rowsC, ssC)

                @pl.when(p < NT - 1)
                def _():
                    s_wait2(dcb, bB, rowsB, ssB)
                    g_start2(icb, bB + 3, rowsB, gsB)

                return carry

            lax.fori_loop(0, NT, triple, 0)
            s_wait2(dcb, SB - 3, rowsA, ssA)
            s_wait2(dcb, SB - 2, rowsB, ssB)
            s_wait2(dcb, SB - 1, rowsC, ssC)

        for hh in range(NST):
            icb, dcb = (idxcb, dstb) if hh % 2 == 0 else (idxcb2, dstb2)
            if hh == 0:
                pltpu.async_copy(idxr.at[c, s, 0], icb, sgI)
                pltpu.async_copy(dstr.at[s, 0], dcb, sgD)
            pltpu.make_async_copy(idxr.at[c, s, hh], icb, sgI).wait()
            pltpu.make_async_copy(dstr.at[s, hh], dcb, sgD).wait()
            if hh + 1 < NST:
                nicb, ndcb = (idxcb2, dstb2) if hh % 2 == 0 else (idxcb, dstb)
                pltpu.async_copy(idxr.at[c, s, hh + 1], nicb, sgI)
                pltpu.async_copy(dstr.at[s, hh + 1], ndcb, sgD)
            run_chunk(icb, dcb)

    scatter_pass(hs2)
    plsc.subcore_barrier()

    b1v = [b1_t[pl.ds(j * 16, 16)] for j in range(8)]

    def epiA_row(r, loc0):
        d = dinv_t[pl.ds(loc0 + r, 16)][0]
        for j in range(8):
            v = rowsA[r, pl.ds(j * 16, 16)]
            v = jnp.maximum(v * d + b1v[j], 0.0) * d
            rowsA[r, pl.ds(j * 16, 16)] = v
        return loc0

    for u in range(10):
        r0 = base + u * 64
        pltpu.sync_copy(acc.at[pl.ds(r0, 64)], rowsA)
        lax.fori_loop(0, 64, epiA_row, u * 64)
        for j in range(4):
            widx[0, pl.ds(j * 16, 16)] = (i16 + (r0 + j * 16)) * 2 + c
        pltpu.sync_copy(rowsA, h1s.at[widx.at[0]])
        for q in range(4):
            pltpu.sync_copy(zrow, acc.at[pl.ds(r0 + q * 16, 16)])

    plsc.subcore_barrier()

    scatter_pass(h1s)
    plsc.subcore_barrier()

    def epiB_grp(g16, loc0):
        rr0 = g16 * 16
        dvec = dinv_t[pl.ds(loc0 + rr0, 16)]
        for k in range(16):
            d = dvec[k]
            r = rr0 + k
            for j in range(8):
                rowsA[r, pl.ds(j * 16, 16)] = rowsA[r, pl.ds(j * 16, 16)] * d
        return loc0

    for u in range(10):
        r0 = base + u * 64
        pltpu.sync_copy(acc.at[pl.ds(r0, 64)], rowsA)
        lax.fori_loop(0, 4, epiB_grp, u * 64)
        pltpu.sync_copy(rowsA, sbkt.at[batch_t.at[u, 0]], add=True)

    plsc.subcore_barrier()

    @pl.when(s == 0)
    def _():
        pltpu.sync_copy(sbkt, ssum.at[c])


_p3 = pl.kernel(
    _p3_body,
    out_type=(_f32((2, GB, HALF)), _f32((NP * 2, HALF))),
    mesh=_mesh,
    scratch_types=[
        pltpu.VMEM((ROWS_T + 16,), jnp.float32),
        pltpu.VMEM((SB, 1, BLK), jnp.int32),
        pltpu.VMEM((SB, 1, BLK), jnp.int32),
        pltpu.VMEM((SB, 1, BLK), jnp.int32),
        pltpu.VMEM((SB, 1, BLK), jnp.int32),
        pltpu.VMEM((1, 64), jnp.int32),
        pltpu.VMEM((BLK, HALF), jnp.float32),
        pltpu.VMEM((BLK, HALF), jnp.float32),
        pltpu.VMEM((BLK, HALF), jnp.float32),
        pltpu.VMEM((16, HALF), jnp.float32),
        pltpu.VMEM((HALF,), jnp.float32),
        pltpu.VMEM((10, 1, 64), jnp.int32),
        pltpu.VMEM_SHARED((NP, HALF), jnp.float32),
        pltpu.VMEM_SHARED((GB, HALF), jnp.float32),
        pltpu.SemaphoreType.DMA,
        pltpu.SemaphoreType.DMA,
        pltpu.SemaphoreType.DMA,
        pltpu.SemaphoreType.DMA,
        pltpu.SemaphoreType.DMA,
        pltpu.SemaphoreType.DMA,
        pltpu.SemaphoreType.DMA,
        pltpu.SemaphoreType.DMA,
    ],
)


def _p4_body(ssum_ref, batch_ref, w2_ref, b2_ref, wp_ref, bp_ref, out_ref):
    batchv = batch_ref[...]
    gids = lax.broadcasted_iota(jnp.int32, (G, NP // 128, 128), 0)
    eq = (batchv[None, :, :] == gids).astype(jnp.float32)
    cnt = jnp.sum(eq, axis=(1, 2))
    S = jnp.concatenate([ssum_ref[0, :G, :], ssum_ref[1, :G, :]], axis=1)
    Sm = S / jnp.maximum(cnt, 1.0)[:, None]
    E2 = jnp.dot(Sm, w2_ref[...], preferred_element_type=jnp.float32) + b2_ref[...]
    out_ref[...] = jnp.dot(E2, wp_ref[...], preferred_element_type=jnp.float32) + bp_ref[...]


def _p4(ssum, batch2d, W2, b2, Wp, bp):
    return pl.pallas_call(
        _p4_body,
        out_shape=_f32((G, 16)),
    )(ssum, batch2d, W2, b2.reshape(1, EMB), Wp, bp.reshape(1, 16))


@jax.jit
def kernel(x, edge_index, batch, W1, b1, W2, b2, Wp, bp):
    src = edge_index[0].astype(jnp.int32)
    dst = edge_index[1].astype(jnp.int32)
    loop = jnp.arange(N, dtype=jnp.int32)
    npad = P_ITEMS - src.shape[0] - N
    fill = jnp.full((npad,), DUMP, jnp.int32)
    src_all = jnp.concatenate([src, loop, fill])
    dst_all = jnp.concatenate([dst, loop, fill])
    src2 = src_all * 2
    idxr = jnp.stack([src2, src2 + 1]).reshape(2, 16, NST, SB, 1, BLK)
    dstr = dst_all.reshape(16, NST, SB, 1, BLK)
    batch_pad = jnp.concatenate(
        [batch.astype(jnp.int32), jnp.full((NP - N,), G, jnp.int32)]
    )
    xpad = jnp.pad(x, ((0, NP - N), (0, 0)))

    degp = _p1(dst_all.reshape(32, BLK_P1, 1, 128))
    Hs, dinv = _p2(xpad, W1, degp.reshape(2, NP, 1))
    hs2 = Hs.reshape(NP * 2, HALF)
    ssum, _h1s = _p3(hs2, idxr, dstr, dinv.reshape(NP),
                     b1.reshape(2, 1, HALF), batch_pad.reshape(16, 10, 1, 64))
    return _p4(ssum, batch_pad.reshape(NP // 128, 128), W2, b2, Wp, bp)
